# Initial kernel scaffold; baseline (speedup 1.0000x reference)
import jax
import jax.numpy as jnp
from jax.experimental import pallas as pl


def kernel(node_features, edge_index, W, b, bn_weight, bn_bias):
    raise NotImplementedError("write your pallas kernel here")


# speedup vs baseline: 7.1083x; 7.1083x over previous
import jax
import jax.numpy as jnp
from jax import lax
from jax.experimental import pallas as pl
from jax.experimental.pallas import tpu as pltpu
from jax.experimental.pallas import tpu_sc as plsc

N = 10000
E = 320000
D = 128
D2 = D // 2
EPS = 1e-5
NEG_SLOPE = 0.01

NC = 2
NS = 16
NW = NC * NS
E_PER_T = E // NS
CHUNK = 80
NCH = E_PER_T // CHUNK
N_PAD = 10240
STRIPE = N_PAD // NS
DEGW = 16
ZROWS = 128
NBLK = STRIPE // ZROWS


def _node_mm_body(x_ref, wd_ref, w2_ref, b_ref, a_ref, bt_ref):
    x = x_ref[...]
    a_ref[...] = jnp.dot(x, wd_ref[...], preferred_element_type=jnp.float32) + b_ref[...]
    bb = jnp.dot(x, w2_ref[...], preferred_element_type=jnp.float32)
    bt_ref[0] = bb[:, :D2]
    bt_ref[1] = bb[:, D2:]


def _node_mm(x, wd_t, w2_t, b2d):
    return pl.pallas_call(
        _node_mm_body,
        out_shape=(
            jax.ShapeDtypeStruct((N, D), jnp.float32),
            jax.ShapeDtypeStruct((NC, N, D2), jnp.float32),
        ),
    )(x, wd_t, w2_t, b2d)


def _edge_scatter_body(src_hbm, dst_hbm, bt_hbm,
                       s_out, d_out,
                       sidx, didx, rows, ones_v, zb_s, zb_d, gsem,
                       acc_s, acc_d):
    c = lax.axis_index("c")
    s = lax.axis_index("s")
    w = c * NS + s

    pltpu.sync_copy(src_hbm.at[s], sidx)
    pltpu.sync_copy(dst_hbm.at[s], didx)

    zero16 = jnp.zeros((16,), jnp.float32)
    one16 = jnp.ones((16,), jnp.float32)

    def fill_zs(i, carry):
        def inner(j, cc):
            zb_s[i, pl.ds(pl.multiple_of(j * 16, 16), 16)] = zero16
            return cc
        return lax.fori_loop(0, D2 // 16, inner, carry)

    lax.fori_loop(0, ZROWS, fill_zs, 0)

    def fill_zd(i, carry):
        def inner(j, cc):
            zb_d[i, j, :] = zero16
            return cc
        return lax.fori_loop(0, ZROWS, inner, carry)

    lax.fori_loop(0, NBLK, fill_zd, 0)

    def fill_on(i, carry):
        ones_v[i, :] = one16
        return carry

    lax.fori_loop(0, CHUNK, fill_on, 0)

    for p in range(NBLK):
        pltpu.sync_copy(zb_s, acc_s.at[pl.ds(s * STRIPE + p * ZROWS, ZROWS)])

    @pl.when(c == 0)
    def _zero_deg():
        for p in range(NBLK):
            pltpu.sync_copy(zb_d.at[p], acc_d.at[pl.ds(s * STRIPE + p * ZROWS, ZROWS)])

    plsc.subcore_barrier()

    def chunk_c0(k, carry):
        pltpu.async_copy(bt_hbm.at[0].at[sidx.at[k]], rows, gsem).wait()
        pltpu.sync_copy(rows, acc_s.at[didx.at[k]], add=True)
        pltpu.sync_copy(ones_v, acc_d.at[didx.at[k]], add=True)
        return carry

    def chunk_c1(k, carry):
        pltpu.async_copy(bt_hbm.at[1].at[sidx.at[k]], rows, gsem).wait()
        pltpu.sync_copy(rows, acc_s.at[didx.at[k]], add=True)
        return carry

    @pl.when(c == 0)
    def _loop0():
        lax.fori_loop(0, NCH, chunk_c0, 0)

    @pl.when(c == 1)
    def _loop1():
        lax.fori_loop(0, NCH, chunk_c1, 0)

    plsc.subcore_barrier()

    for p in range(NBLK):
        pltpu.sync_copy(acc_s.at[pl.ds(s * STRIPE + p * ZROWS, ZROWS)], zb_s)
        pltpu.sync_copy(zb_s, s_out.at[w, p])

    @pl.when(c == 0)
    def _dump_deg():
        for p in range(NBLK):
            pltpu.sync_copy(acc_d.at[pl.ds(s * STRIPE + p * ZROWS, ZROWS)], zb_d.at[p])
            pltpu.sync_copy(zb_d.at[p], d_out.at[s, p])


def _edge_scatter(src3d, dst3d, bt):
    mesh = plsc.VectorSubcoreMesh(core_axis_name="c", subcore_axis_name="s")
    k = pl.kernel(
        _edge_scatter_body,
        out_type=(
            jax.ShapeDtypeStruct((NW, NBLK, ZROWS, D2), jnp.float32),
            jax.ShapeDtypeStruct((NS, NBLK, ZROWS, DEGW), jnp.float32),
        ),
        mesh=mesh,
        compiler_params=pltpu.CompilerParams(use_tc_tiling_on_sc=False),
        scratch_types=[
            pltpu.VMEM((NCH, CHUNK), jnp.int32),
            pltpu.VMEM((NCH, CHUNK), jnp.int32),
            pltpu.VMEM((CHUNK, D2), jnp.float32),
            pltpu.VMEM((CHUNK, DEGW), jnp.float32),
            pltpu.VMEM((ZROWS, D2), jnp.float32),
            pltpu.VMEM((NBLK, ZROWS, DEGW), jnp.float32),
            pltpu.SemaphoreType.DMA,
            pltpu.VMEM_SHARED((N_PAD, D2), jnp.float32),
            pltpu.VMEM_SHARED((N_PAD, DEGW), jnp.float32),
        ],
    )
    return k(src3d, dst3d, bt)


def _finalize_body(a_ref, s0_ref, s1_ref, d_ref, g_ref, be_ref, o_ref):
    s = jnp.concatenate([s0_ref[...], s1_ref[...]], axis=-1)
    deg = d_ref[:, 0:1]
    h = deg * a_ref[...] + s
    mean = jnp.mean(h, axis=0, keepdims=True)
    var = jnp.mean((h - mean) ** 2, axis=0, keepdims=True)
    hn = (h - mean) * lax.rsqrt(var + EPS) * g_ref[...] + be_ref[...]
    o_ref[...] = jnp.where(hn >= 0, hn, NEG_SLOPE * hn)


def _finalize(a, s0, s1, d, gamma2d, beta2d):
    return pl.pallas_call(
        _finalize_body,
        out_shape=jax.ShapeDtypeStruct((N, D), jnp.float32),
    )(a, s0, s1, d, gamma2d, beta2d)


def kernel(node_features, edge_index, W, b, bn_weight, bn_bias):
    x = node_features.astype(jnp.float32)
    w1t = W[:, :D].T
    w2t = W[:, D:].T
    wd_t = w1t - w2t

    a, bt = _node_mm(x, wd_t, w2t, jnp.broadcast_to(b[None, :], (1, D)))

    src = edge_index[0].astype(jnp.int32).reshape(NS, NCH, CHUNK)
    dst = edge_index[1].astype(jnp.int32).reshape(NS, NCH, CHUNK)

    s_part, d_part = _edge_scatter(src, dst, bt)

    s0 = s_part[:NS].reshape(N_PAD, D2)[:N]
    s1 = s_part[NS:].reshape(N_PAD, D2)[:N]
    d = d_part.reshape(N_PAD, DEGW)[:N]

    return _finalize(a, s0, s1, d,
                     jnp.broadcast_to(bn_weight[None, :], (1, D)),
                     jnp.broadcast_to(bn_bias[None, :], (1, D)))


# speedup vs baseline: 10.8281x; 1.5233x over previous
import jax
import jax.numpy as jnp
from jax import lax
from jax.experimental import pallas as pl
from jax.experimental.pallas import tpu as pltpu
from jax.experimental.pallas import tpu_sc as plsc

N = 10000
E = 320000
D = 128
D2 = D // 2
EPS = 1e-5
NEG_SLOPE = 0.01

NC = 2
NS = 16
NW = NC * NS
E_PER_T = E // NS
CHUNK = 80
NCH = E_PER_T // CHUNK
N_PAD = 10240
STRIPE = N_PAD // NS
DEGW = 16
ZROWS = 128
NBLK = STRIPE // ZROWS


def _node_mm_body(x_ref, wd_ref, w2_ref, b_ref, a_ref, bt_ref):
    x = x_ref[...]
    a_ref[...] = jnp.dot(x, wd_ref[...], preferred_element_type=jnp.float32) + b_ref[...]
    bb = jnp.dot(x, w2_ref[...], preferred_element_type=jnp.float32)
    bt_ref[0] = bb[:, :D2]
    bt_ref[1] = bb[:, D2:]


def _node_mm(x, wd_t, w2_t, b2d):
    return pl.pallas_call(
        _node_mm_body,
        out_shape=(
            jax.ShapeDtypeStruct((N, D), jnp.float32),
            jax.ShapeDtypeStruct((NC, N, D2), jnp.float32),
        ),
    )(x, wd_t, w2_t, b2d)


def _edge_scatter_body(src_hbm, dst_hbm, bt_hbm,
                       s_out, d_out,
                       sidx, didx, rows0, rows1, ones_v, zb_s, zb_d,
                       gsem0, gsem1,
                       acc_s, acc_d):
    c = lax.axis_index("c")
    s = lax.axis_index("s")
    w = c * NS + s

    pltpu.sync_copy(src_hbm.at[s], sidx)
    pltpu.sync_copy(dst_hbm.at[s], didx)

    zero16 = jnp.zeros((16,), jnp.float32)
    one16 = jnp.ones((16,), jnp.float32)

    def fill_zs(i, carry):
        def inner(j, cc):
            zb_s[i, pl.ds(pl.multiple_of(j * 16, 16), 16)] = zero16
            return cc
        return lax.fori_loop(0, D2 // 16, inner, carry)

    lax.fori_loop(0, ZROWS, fill_zs, 0)

    def fill_zd(i, carry):
        def inner(j, cc):
            zb_d[i, j, :] = zero16
            return cc
        return lax.fori_loop(0, ZROWS, inner, carry)

    lax.fori_loop(0, NBLK, fill_zd, 0)

    def fill_on(i, carry):
        ones_v[i, :] = one16
        return carry

    lax.fori_loop(0, CHUNK, fill_on, 0)

    for p in range(NBLK):
        pltpu.sync_copy(zb_s, acc_s.at[pl.ds(s * STRIPE + p * ZROWS, ZROWS)])
        pltpu.sync_copy(zb_d.at[p], acc_d.at[pl.ds(s * STRIPE + p * ZROWS, ZROWS)])

    plsc.subcore_barrier()

    def make_loop(half, deg_lo):
        rows_b = (rows0, rows1)
        sem_b = (gsem0, gsem1)

        def pair(g, carry):
            for b in range(2):
                k = 2 * g + b
                pltpu.make_async_copy(bt_hbm.at[half].at[sidx.at[k]],
                                      rows_b[b], sem_b[b]).wait()
                pltpu.sync_copy(rows_b[b], acc_s.at[didx.at[k]], add=True)

                @pl.when((k >= deg_lo) & (k < deg_lo + NCH // 2))
                def _deg():
                    pltpu.sync_copy(ones_v, acc_d.at[didx.at[k]], add=True)

                @pl.when(g < NCH // 2 - 1)
                def _prefetch():
                    pltpu.async_copy(bt_hbm.at[half].at[sidx.at[k + 2]],
                                     rows_b[b], sem_b[b])
            return carry

        def run():
            pltpu.async_copy(bt_hbm.at[half].at[sidx.at[0]], rows0, gsem0)
            pltpu.async_copy(bt_hbm.at[half].at[sidx.at[1]], rows1, gsem1)
            lax.fori_loop(0, NCH // 2, pair, 0)

        return run

    pl.when(c == 0)(make_loop(0, 0))
    pl.when(c == 1)(make_loop(1, NCH // 2))

    plsc.subcore_barrier()

    for p in range(NBLK):
        pltpu.sync_copy(acc_s.at[pl.ds(s * STRIPE + p * ZROWS, ZROWS)], zb_s)
        pltpu.sync_copy(zb_s, s_out.at[w, p])

    for p in range(NBLK):
        pltpu.sync_copy(acc_d.at[pl.ds(s * STRIPE + p * ZROWS, ZROWS)], zb_d.at[p])
        pltpu.sync_copy(zb_d.at[p], d_out.at[w, p])


def _edge_scatter(src3d, dst3d, bt):
    mesh = plsc.VectorSubcoreMesh(core_axis_name="c", subcore_axis_name="s")
    k = pl.kernel(
        _edge_scatter_body,
        out_type=(
            jax.ShapeDtypeStruct((NW, NBLK, ZROWS, D2), jnp.float32),
            jax.ShapeDtypeStruct((NW, NBLK, ZROWS, DEGW), jnp.float32),
        ),
        mesh=mesh,
        compiler_params=pltpu.CompilerParams(use_tc_tiling_on_sc=False),
        scratch_types=[
            pltpu.VMEM((NCH, CHUNK), jnp.int32),
            pltpu.VMEM((NCH, CHUNK), jnp.int32),
            pltpu.VMEM((CHUNK, D2), jnp.float32),
            pltpu.VMEM((CHUNK, D2), jnp.float32),
            pltpu.VMEM((CHUNK, DEGW), jnp.float32),
            pltpu.VMEM((ZROWS, D2), jnp.float32),
            pltpu.VMEM((NBLK, ZROWS, DEGW), jnp.float32),
            pltpu.SemaphoreType.DMA,
            pltpu.SemaphoreType.DMA,
            pltpu.VMEM_SHARED((N_PAD, D2), jnp.float32),
            pltpu.VMEM_SHARED((N_PAD, DEGW), jnp.float32),
        ],
    )
    return k(src3d, dst3d, bt)


def _finalize_body(a_ref, s0_ref, s1_ref, d0_ref, d1_ref, g_ref, be_ref, o_ref):
    s = jnp.concatenate([s0_ref[...], s1_ref[...]], axis=-1)
    deg = d0_ref[:, 0:1] + d1_ref[:, 0:1]
    h = deg * a_ref[...] + s
    mean = jnp.mean(h, axis=0, keepdims=True)
    var = jnp.mean((h - mean) ** 2, axis=0, keepdims=True)
    hn = (h - mean) * lax.rsqrt(var + EPS) * g_ref[...] + be_ref[...]
    o_ref[...] = jnp.where(hn >= 0, hn, NEG_SLOPE * hn)


def _finalize(a, s0, s1, d0, d1, gamma2d, beta2d):
    return pl.pallas_call(
        _finalize_body,
        out_shape=jax.ShapeDtypeStruct((N, D), jnp.float32),
    )(a, s0, s1, d0, d1, gamma2d, beta2d)


def kernel(node_features, edge_index, W, b, bn_weight, bn_bias):
    x = node_features.astype(jnp.float32)
    w1t = W[:, :D].T
    w2t = W[:, D:].T
    wd_t = w1t - w2t

    a, bt = _node_mm(x, wd_t, w2t, jnp.broadcast_to(b[None, :], (1, D)))

    src = edge_index[0].astype(jnp.int32).reshape(NS, NCH, CHUNK)
    dst = edge_index[1].astype(jnp.int32).reshape(NS, NCH, CHUNK)

    s_part, d_part = _edge_scatter(src, dst, bt)

    s0 = s_part[:NS].reshape(N_PAD, D2)[:N]
    s1 = s_part[NS:].reshape(N_PAD, D2)[:N]
    d0 = d_part[:NS].reshape(N_PAD, DEGW)[:N]
    d1 = d_part[NS:].reshape(N_PAD, DEGW)[:N]

    return _finalize(a, s0, s1, d0, d1,
                     jnp.broadcast_to(bn_weight[None, :], (1, D)),
                     jnp.broadcast_to(bn_bias[None, :], (1, D)))


# speedup vs baseline: 11.9120x; 1.1001x over previous
import jax
import jax.numpy as jnp
from jax import lax
from jax.experimental import pallas as pl
from jax.experimental.pallas import tpu as pltpu
from jax.experimental.pallas import tpu_sc as plsc

N = 10000
E = 320000
D = 128
D2 = D // 2
EPS = 1e-5
NEG_SLOPE = 0.01

NC = 2
NS = 16
NW = NC * NS
E_PER_T = E // NS
CHUNK = 80
NCH = E_PER_T // CHUNK
N_PAD = 10240
STRIPE = N_PAD // NS
DEGW = 16
ZROWS = 80
NBLK = STRIPE // ZROWS


def _node_mm_body(x_ref, wd_ref, w2_ref, b_ref, a_ref, bt_ref):
    x = x_ref[...]
    a_ref[...] = jnp.dot(x, wd_ref[...], preferred_element_type=jnp.float32) + b_ref[...]
    bb = jnp.dot(x, w2_ref[...], preferred_element_type=jnp.float32)
    bt_ref[0] = bb[:, :D2]
    bt_ref[1] = bb[:, D2:]


def _node_mm(x, wd_t, w2_t, b2d):
    return pl.pallas_call(
        _node_mm_body,
        out_shape=(
            jax.ShapeDtypeStruct((N, D), jnp.float32),
            jax.ShapeDtypeStruct((NC, N, D2), jnp.float32),
        ),
    )(x, wd_t, w2_t, b2d)


NBUF = 5
PREF = 2


def _edge_scatter_body(src_hbm, dst_hbm, bt_hbm,
                       s_out, d_out,
                       sidx, didx, rows0, rows1, rows2, rows3, rows4,
                       ones_v, zb_d,
                       gsem0, gsem1, gsem2, gsem3, gsem4,
                       ssem0, ssem1, ssem2, ssem3, ssem4, osem,
                       acc_s, acc_d):
    c = lax.axis_index("c")
    s = lax.axis_index("s")
    w = c * NS + s

    pltpu.sync_copy(src_hbm.at[s], sidx)
    pltpu.sync_copy(dst_hbm.at[s], didx)

    zero16 = jnp.zeros((16,), jnp.float32)
    one16 = jnp.ones((16,), jnp.float32)

    def fill_zs(i, carry):
        def inner(j, cc):
            rows0[i, pl.ds(pl.multiple_of(j * 16, 16), 16)] = zero16
            return cc
        return lax.fori_loop(0, D2 // 16, inner, carry)

    lax.fori_loop(0, ZROWS, fill_zs, 0)

    def fill_zd(i, carry):
        zb_d[i, :] = zero16
        return carry

    lax.fori_loop(0, ZROWS, fill_zd, 0)

    def fill_on(i, carry):
        ones_v[i, :] = one16
        return carry

    lax.fori_loop(0, CHUNK, fill_on, 0)

    for p in range(NBLK):
        pltpu.sync_copy(rows0, acc_s.at[pl.ds(s * STRIPE + p * ZROWS, ZROWS)])
        pltpu.sync_copy(zb_d, acc_d.at[pl.ds(s * STRIPE + p * ZROWS, ZROWS)])

    plsc.subcore_barrier()

    rows_b = (rows0, rows1, rows2, rows3, rows4)
    gsems = (gsem0, gsem1, gsem2, gsem3, gsem4)
    ssems = (ssem0, ssem1, ssem2, ssem3, ssem4)
    LAG = NBUF - PREF

    def make_loop(half, deg_lo):
        def visit(k, b):
            pltpu.make_async_copy(bt_hbm.at[half].at[sidx.at[k]],
                                  rows_b[b], gsems[b]).wait()
            pltpu.async_copy(rows_b[b], acc_s.at[didx.at[k]], ssems[b],
                             add=True)

            @pl.when((k >= deg_lo) & (k < deg_lo + NCH // 2))
            def _deg():
                pltpu.async_copy(ones_v, acc_d.at[didx.at[k]], osem, add=True)

            bn = (b + PREF) % NBUF

            @pl.when(k >= LAG)
            def _free():
                pltpu.make_async_copy(rows_b[bn], acc_s.at[didx.at[0]],
                                      ssems[bn]).wait()

            @pl.when(k + PREF < NCH)
            def _prefetch():
                pltpu.async_copy(bt_hbm.at[half].at[sidx.at[k + PREF]],
                                 rows_b[bn], gsems[bn])

        def group(g, carry):
            for b in range(NBUF):
                visit(NBUF * g + b, b)
            return carry

        def run():
            for b in range(PREF):
                pltpu.async_copy(bt_hbm.at[half].at[sidx.at[b]],
                                 rows_b[b], gsems[b])
            lax.fori_loop(0, NCH // NBUF, group, 0)

        return run

    pl.when(c == 0)(make_loop(0, 0))
    pl.when(c == 1)(make_loop(1, NCH // 2))

    for k in range(NCH - LAG, NCH):
        b = k % NBUF
        pltpu.make_async_copy(rows_b[b], acc_s.at[didx.at[0]],
                              ssems[b]).wait()

    def drain_ones(i, carry):
        pltpu.make_async_copy(ones_v, acc_d.at[didx.at[0]], osem).wait()
        return carry

    lax.fori_loop(0, NCH // 2, drain_ones, 0)

    plsc.subcore_barrier()

    for p in range(NBLK):
        pltpu.sync_copy(acc_s.at[pl.ds(s * STRIPE + p * ZROWS, ZROWS)], rows0)
        pltpu.sync_copy(rows0, s_out.at[w, p])
        pltpu.sync_copy(acc_d.at[pl.ds(s * STRIPE + p * ZROWS, ZROWS)], zb_d)
        pltpu.sync_copy(zb_d, d_out.at[w, p])


def _edge_scatter(src3d, dst3d, bt):
    mesh = plsc.VectorSubcoreMesh(core_axis_name="c", subcore_axis_name="s")
    k = pl.kernel(
        _edge_scatter_body,
        out_type=(
            jax.ShapeDtypeStruct((NW, NBLK, ZROWS, D2), jnp.float32),
            jax.ShapeDtypeStruct((NW, NBLK, ZROWS, DEGW), jnp.float32),
        ),
        mesh=mesh,
        compiler_params=pltpu.CompilerParams(use_tc_tiling_on_sc=False),
        scratch_types=[
            pltpu.VMEM((NCH, CHUNK), jnp.int32),
            pltpu.VMEM((NCH, CHUNK), jnp.int32),
            pltpu.VMEM((CHUNK, D2), jnp.float32),
            pltpu.VMEM((CHUNK, D2), jnp.float32),
            pltpu.VMEM((CHUNK, D2), jnp.float32),
            pltpu.VMEM((CHUNK, D2), jnp.float32),
            pltpu.VMEM((CHUNK, D2), jnp.float32),
            pltpu.VMEM((CHUNK, DEGW), jnp.float32),
            pltpu.VMEM((ZROWS, DEGW), jnp.float32),
            pltpu.SemaphoreType.DMA,
            pltpu.SemaphoreType.DMA,
            pltpu.SemaphoreType.DMA,
            pltpu.SemaphoreType.DMA,
            pltpu.SemaphoreType.DMA,
            pltpu.SemaphoreType.DMA,
            pltpu.SemaphoreType.DMA,
            pltpu.SemaphoreType.DMA,
            pltpu.SemaphoreType.DMA,
            pltpu.SemaphoreType.DMA,
            pltpu.SemaphoreType.DMA,
            pltpu.VMEM_SHARED((N_PAD, D2), jnp.float32),
            pltpu.VMEM_SHARED((N_PAD, DEGW), jnp.float32),
        ],
    )
    return k(src3d, dst3d, bt)


def _finalize_body(a_ref, s0_ref, s1_ref, d0_ref, d1_ref, g_ref, be_ref, o_ref):
    s = jnp.concatenate([s0_ref[...], s1_ref[...]], axis=-1)
    deg = d0_ref[:, 0:1] + d1_ref[:, 0:1]
    h = deg * a_ref[...] + s
    mean = jnp.mean(h, axis=0, keepdims=True)
    var = jnp.mean((h - mean) ** 2, axis=0, keepdims=True)
    hn = (h - mean) * lax.rsqrt(var + EPS) * g_ref[...] + be_ref[...]
    o_ref[...] = jnp.where(hn >= 0, hn, NEG_SLOPE * hn)


def _finalize(a, s0, s1, d0, d1, gamma2d, beta2d):
    return pl.pallas_call(
        _finalize_body,
        out_shape=jax.ShapeDtypeStruct((N, D), jnp.float32),
    )(a, s0, s1, d0, d1, gamma2d, beta2d)


def kernel(node_features, edge_index, W, b, bn_weight, bn_bias):
    x = node_features.astype(jnp.float32)
    w1t = W[:, :D].T
    w2t = W[:, D:].T
    wd_t = w1t - w2t

    a, bt = _node_mm(x, wd_t, w2t, jnp.broadcast_to(b[None, :], (1, D)))

    src = edge_index[0].astype(jnp.int32).reshape(NS, NCH, CHUNK)
    dst = edge_index[1].astype(jnp.int32).reshape(NS, NCH, CHUNK)

    s_part, d_part = _edge_scatter(src, dst, bt)

    s0 = s_part[:NS].reshape(N_PAD, D2)[:N]
    s1 = s_part[NS:].reshape(N_PAD, D2)[:N]
    d0 = d_part[:NS].reshape(N_PAD, DEGW)[:N]
    d1 = d_part[NS:].reshape(N_PAD, DEGW)[:N]

    return _finalize(a, s0, s1, d0, d1,
                     jnp.broadcast_to(bn_weight[None, :], (1, D)),
                     jnp.broadcast_to(bn_bias[None, :], (1, D)))
